# baseline (speedup 1.0000x reference)
import numpy as np
import jax
import jax.numpy as jnp
from jax.experimental import pallas as pl

_ROWS = 128


def kernel(logits):
    s = jnp.sum(logits, axis=-1)
    return s.astype(jnp.int32), s, s
